# baseline (speedup 1.0000x reference)
import functools

import jax
import jax.numpy as jnp
from jax import lax
from jax.experimental import pallas as pl
from jax.experimental.pallas import tpu as pltpu
from jax.experimental.pallas import tpu_sc as plsc

F32 = jnp.float32
I32 = jnp.int32

NTILES = 32
RPT = 320
PN = NTILES * RPT
CHUNK = 4000
LANES = 16


def _mesh():
    return plsc.VectorSubcoreMesh(core_axis_name="c", subcore_axis_name="s")


def _wid():
    return lax.axis_index("c") * 16 + lax.axis_index("s")


_DSL = 128


def _make_sc_deg(E):
    ept = E // NTILES
    nsl = ept // _DSL
    tail = ept - nsl * _DSL
    nslp = nsl + (1 if tail else 0)

    @functools.partial(
        pl.kernel,
        out_type=(jax.ShapeDtypeStruct((PN,), F32),
                  jax.ShapeDtypeStruct((PN,), F32)),
        mesh=_mesh(),
        compiler_params=pltpu.CompilerParams(needs_layout_passes=False, use_tc_tiling_on_sc=False),
        scratch_types=[
            pltpu.VMEM((nslp, _DSL), I32),
            pltpu.VMEM((nslp, _DSL), F32),
            pltpu.VMEM((1024,), F32),
            pltpu.VMEM_SHARED((PN,), F32),
            pltpu.SemaphoreType.DMA,
            pltpu.SemaphoreType.DMA,
            pltpu.SemaphoreType.DMA,
        ],
    )
    def sc_deg(ei_hbm, ea_hbm, deg0_hbm, deg1_hbm, dstb, eab, zbuf, sdeg,
               seml, semz, sems):
        cid = lax.axis_index("c")
        sid = lax.axis_index("s")
        wid = cid * 16 + sid
        off = wid * ept

        if tail:
            def _zt(i, _):
                dstb[nsl, pl.ds(i * LANES, LANES)] = jnp.zeros((LANES,), I32)
                eab[nsl, pl.ds(i * LANES, LANES)] = jnp.zeros((LANES,), F32)
                return 0
            lax.fori_loop(0, _DSL // LANES, _zt, 0)

        def _ld(j, _):
            pltpu.async_copy(ei_hbm.at[1, pl.ds(off + j * _DSL, _DSL)], dstb.at[j], seml)
            pltpu.async_copy(ea_hbm.at[pl.ds(off + j * _DSL, _DSL)], eab.at[j], seml)
            return 0
        lax.fori_loop(0, nsl, _ld, 0)
        if tail:
            pltpu.async_copy(ei_hbm.at[1, pl.ds(off + nsl * _DSL, tail)],
                             dstb.at[nsl, pl.ds(0, tail)], seml)
            pltpu.async_copy(ea_hbm.at[pl.ds(off + nsl * _DSL, tail)],
                             eab.at[nsl, pl.ds(0, tail)], seml)

        def _z(i, _):
            zbuf[pl.ds(i * LANES, LANES)] = jnp.zeros((LANES,), F32)
            return 0
        lax.fori_loop(0, 1024 // LANES, _z, 0)

        @pl.when(sid == 0)
        def _():
            def _zs(k, _):
                pltpu.async_copy(zbuf, sdeg.at[pl.ds(k * 1024, 1024)], semz)
                return 0
            lax.fori_loop(0, PN // 1024, _zs, 0)

            def _zw(k, _):
                pltpu.make_async_copy(zbuf, sdeg.at[pl.ds(k * 1024, 1024)], semz).wait()
                return 0
            lax.fori_loop(0, PN // 1024, _zw, 0)
        plsc.subcore_barrier()

        def _lw(j, _):
            pltpu.make_async_copy(ei_hbm.at[1, pl.ds(off + j * _DSL, _DSL)], dstb.at[j], seml).wait()
            pltpu.make_async_copy(ea_hbm.at[pl.ds(off + j * _DSL, _DSL)], eab.at[j], seml).wait()
            return 0
        lax.fori_loop(0, nsl, _lw, 0)
        if tail:
            pltpu.make_async_copy(ei_hbm.at[1, pl.ds(off + nsl * _DSL, tail)],
                                  dstb.at[nsl, pl.ds(0, tail)], seml).wait()
            pltpu.make_async_copy(ea_hbm.at[pl.ds(off + nsl * _DSL, tail)],
                                  eab.at[nsl, pl.ds(0, tail)], seml).wait()

        def _sc(j, _):
            pltpu.async_copy(eab.at[j], sdeg.at[dstb.at[j]], sems, add=True)
            return 0
        lax.fori_loop(0, nslp, _sc, 0)

        def _sw(j, _):
            pltpu.make_async_copy(eab.at[j], sdeg.at[dstb.at[j]], sems).wait()
            return 0
        lax.fori_loop(0, nslp, _sw, 0)

        plsc.subcore_barrier()
        @pl.when(sid == 0)
        def _():
            @pl.when(cid == 0)
            def _():
                pltpu.sync_copy(sdeg, deg0_hbm)

            @pl.when(cid == 1)
            def _():
                pltpu.sync_copy(sdeg, deg1_hbm)

    return sc_deg


_GB = 128


def _make_sc_agg(E):
    nch = E // CHUNK
    nvec = CHUNK // LANES
    ccap = CHUNK + 2 * _GB

    @functools.partial(
        pl.kernel,
        out_type=jax.ShapeDtypeStruct((PN, 128), F32),
        mesh=_mesh(),
        compiler_params=pltpu.CompilerParams(needs_layout_passes=False, use_tc_tiling_on_sc=False),
        scratch_types=[
            pltpu.VMEM((PN + LANES,), F32),
            pltpu.VMEM((2, CHUNK), I32),
            pltpu.VMEM((2, CHUNK), I32),
            pltpu.VMEM((2, CHUNK), F32),
            pltpu.VMEM((ccap,), I32),
            pltpu.VMEM((ccap,), I32),
            pltpu.VMEM((ccap,), F32),
            pltpu.VMEM((LANES, 128), F32),
            pltpu.VMEM((2, _GB, 128), F32),
            pltpu.VMEM((RPT, 128), F32),
            pltpu.SemaphoreType.DMA((2,)),
            pltpu.SemaphoreType.DMA((2,)),
            pltpu.SemaphoreType.DMA,
        ],
    )
    def sc_agg(ei_hbm, ea_hbm, dinv_hbm, h96_hbm, acc_hbm,
               dinv_v, dstb, srcb, eab, src_c, dl_c, nrm_c,
               gbuf, gb2, acc, sems, semg, semt):
        wid = _wid()
        lo = wid * RPT

        def _fire_scan(c):
            p = c % 2
            off = c * CHUNK
            pltpu.async_copy(ei_hbm.at[1, pl.ds(off, CHUNK)], dstb.at[p], sems.at[p])
            pltpu.async_copy(ei_hbm.at[0, pl.ds(off, CHUNK)], srcb.at[p], sems.at[p])
            pltpu.async_copy(ea_hbm.at[pl.ds(off, CHUNK)], eab.at[p], sems.at[p])

        _fire_scan(0)
        pltpu.sync_copy(dinv_hbm, dinv_v.at[pl.ds(0, PN)])

        def _z(i, _):
            src_c[pl.ds(i * LANES, LANES)] = jnp.zeros((LANES,), I32)
            dl_c[pl.ds(i * LANES, LANES)] = jnp.zeros((LANES,), I32)
            return 0
        lax.fori_loop(0, ccap // LANES, _z, 0)

        for base, blk in ((0, _GB), (_GB, _GB), (2 * _GB, RPT - 2 * _GB)):
            pltpu.sync_copy(h96_hbm.at[pl.ds(lo + base, blk)],
                            gb2.at[0, pl.ds(0, blk)])

            def _init(r, _, base=base):
                dv = dinv_v[pl.ds(lo + base + r, LANES)][0]
                d2 = dv * dv
                for k in range(6):
                    acc[base + r, pl.ds(16 * k, 16)] = \
                        d2 * gb2[0, r, pl.ds(16 * k, 16)]
                return 0
            lax.fori_loop(0, blk, _init, 0)

        def _drain_scan(c):
            p = c % 2
            off = c * CHUNK
            pltpu.make_async_copy(ei_hbm.at[1, pl.ds(off, CHUNK)], dstb.at[p], sems.at[p]).wait()
            pltpu.make_async_copy(ei_hbm.at[0, pl.ds(off, CHUNK)], srcb.at[p], sems.at[p]).wait()
            pltpu.make_async_copy(ea_hbm.at[pl.ds(off, CHUNK)], eab.at[p], sems.at[p]).wait()

        def _fire_gather(b):
            p = b % 2
            idxs = src_c.at[pl.ds(b * _GB, _GB)]
            pltpu.async_copy(h96_hbm.at[idxs], gb2.at[p], semg.at[p])

        def _drain_gather(b):
            p = b % 2
            pltpu.make_async_copy(h96_hbm.at[src_c.at[pl.ds(0, _GB)]],
                                  gb2.at[p], semg.at[p]).wait()

        def _edge_update(nrm, dl, row_k):
            for k in range(3):
                o = 16 * k
                am = acc[dl, pl.ds(o, 16)]
                acc[dl, pl.ds(o, 16)] = jnp.maximum(am, nrm * row_k(o))
            for k in range(3):
                o = 48 + 16 * k
                aa = acc[dl, pl.ds(o, 16)]
                acc[dl, pl.ds(o, 16)] = aa + nrm * row_k(o)

        def _chunk(c, cnt_in):
            p = c % 2

            @pl.when(c + 1 < nch)
            def _():
                _fire_scan(c + 1)
            _drain_scan(c)

            def _scan(i, carry):
                cnt, fired = carry
                for u in range(2):
                    o = (i * 2 + u) * LANES
                    d = dstb[p, pl.ds(o, LANES)]
                    s = srcb[p, pl.ds(o, LANES)]
                    w = eab[p, pl.ds(o, LANES)]
                    m = (d >= lo) & (d < lo + RPT)
                    plsc.store_compressed(src_c.at[pl.ds(cnt, LANES)], s, mask=m)
                    plsc.store_compressed(dl_c.at[pl.ds(cnt, LANES)], d - lo, mask=m)
                    plsc.store_compressed(nrm_c.at[pl.ds(cnt, LANES)], w, mask=m)
                    cnt = cnt + jnp.sum(m.astype(I32))

                @pl.when((cnt >= _GB) & (fired == 0))
                def _():
                    _fire_gather(0)
                fired = jnp.where(cnt >= _GB, 1, fired)
                return cnt, fired
            cnt, _fired = lax.fori_loop(0, nvec // 2, _scan, (cnt_in, 0))

            full = cnt // _GB

            def _batch(b, _):
                bp = b % 2

                @pl.when(b + 1 < full)
                def _():
                    _fire_gather(b + 1)
                _drain_gather(b)

                def _grp(g, _):
                    e0 = b * _GB + g * LANES
                    sv = src_c[pl.ds(e0, LANES)]
                    dv = dl_c[pl.ds(e0, LANES)]
                    wv = nrm_c[pl.ds(e0, LANES)]
                    nv = (plsc.load_gather(dinv_v, [sv]) * wv
                          * plsc.load_gather(dinv_v, [dv + lo]))
                    for j in range(LANES):
                        _edge_update(nv[j], dv[j],
                                     lambda o, j=j: gb2[bp, g * LANES + j,
                                                        pl.ds(o, 16)])
                    return 0
                lax.fori_loop(0, _GB // LANES, _grp, 0)
                return 0
            lax.fori_loop(0, full, _batch, 0)

            rem = cnt - full * _GB

            @pl.when(full > 0)
            def _():
                done = full * _GB
                for i in range(_GB // LANES):
                    o = i * LANES
                    src_c[pl.ds(o, LANES)] = src_c[pl.ds(done + o, LANES)]
                    dl_c[pl.ds(o, LANES)] = dl_c[pl.ds(done + o, LANES)]
                    nrm_c[pl.ds(o, LANES)] = nrm_c[pl.ds(done + o, LANES)]
            return rem
        rem = lax.fori_loop(0, nch, _chunk, 0)

        def _tailg(g, _):
            idxv = src_c[pl.ds(g * LANES, LANES)]
            pltpu.async_copy(h96_hbm.at[idxv], gbuf, semt).wait()
            ecnt = jnp.minimum(LANES, rem - g * LANES)
            dv = dl_c[pl.ds(g * LANES, LANES)]
            wv = nrm_c[pl.ds(g * LANES, LANES)]
            nv = (plsc.load_gather(dinv_v, [idxv]) * wv
                  * plsc.load_gather(dinv_v, [dv + lo]))
            nrm_c[pl.ds(g * LANES, LANES)] = nv

            def _edge(j, _):
                e = g * LANES + j
                nrm = nrm_c[pl.ds(e, LANES)][0]
                dl = dl_c[pl.ds(e, LANES)][0]
                _edge_update(nrm, dl, lambda o: gbuf[j, pl.ds(o, 16)])
                return 0
            lax.fori_loop(0, ecnt, _edge, 0)
            return 0
        lax.fori_loop(0, (rem + LANES - 1) // LANES, _tailg, 0)

        pltpu.sync_copy(acc, acc_hbm.at[pl.ds(lo, RPT)])

    return sc_agg


_BN = 1024


def _tc_h_body(x_ref, w96_ref, ws_ref, bs_ref, h96_ref, xs_ref):
    xb = x_ref[...]
    h96_ref[...] = lax.dot_general(xb, w96_ref[...], (((1,), (1,)), ((), ())),
                                   preferred_element_type=F32)
    xs = lax.dot_general(xb, ws_ref[...], (((1,), (1,)), ((), ())),
                         preferred_element_type=F32)
    xs_ref[...] = jnp.maximum(xs + bs_ref[...], 0.0)


def _tc_h(x_pad, w96, ws, bs2):
    grid = (PN // _BN,)
    return pl.pallas_call(
        _tc_h_body,
        grid=grid,
        in_specs=[
            pl.BlockSpec((_BN, x_pad.shape[1]), lambda i: (i, 0)),
            pl.BlockSpec(w96.shape, lambda i: (0, 0)),
            pl.BlockSpec(ws.shape, lambda i: (0, 0)),
            pl.BlockSpec(bs2.shape, lambda i: (0, 0)),
        ],
        out_specs=[
            pl.BlockSpec((_BN, 128), lambda i: (i, 0)),
            pl.BlockSpec((_BN, 48), lambda i: (i, 0)),
        ],
        out_shape=[
            jax.ShapeDtypeStruct((PN, 128), F32),
            jax.ShapeDtypeStruct((PN, 48), F32),
        ],
    )(x_pad, w96, ws, bs2)


def _tc_dinv_body(d0_ref, d1_ref, dinv_ref):
    dinv_ref[...] = lax.rsqrt(d0_ref[...] + d1_ref[...] + 1.0)


def _tc_dinv(deg0, deg1):
    grid = (PN // _BN,)
    spec = pl.BlockSpec((1, 1, _BN), lambda i: (i, 0, 0))
    return pl.pallas_call(
        _tc_dinv_body,
        grid=grid,
        in_specs=[spec, spec],
        out_specs=spec,
        out_shape=jax.ShapeDtypeStruct((PN // _BN, 1, _BN), F32),
    )(deg0, deg1)


def _tc_mlp_body(acc_ref, xs_ref, bm_ref, ba_ref,
                 w1m_ref, w1a_ref, w1s_ref, b1_ref, w2_ref, b2_ref, out_ref):
    xm = jnp.maximum(acc_ref[:, 0:48] + bm_ref[...], 0.0)
    xa = jnp.maximum(acc_ref[:, 48:96] + ba_ref[...], 0.0)
    xs = xs_ref[...]
    dn = (((1,), (0,)), ((), ()))
    h1 = lax.dot_general(xm, w1m_ref[...], dn, preferred_element_type=F32)
    h1 = h1 + lax.dot_general(xa, w1a_ref[...], dn, preferred_element_type=F32)
    h1 = h1 + lax.dot_general(xs, w1s_ref[...], dn, preferred_element_type=F32)
    h1 = jnp.maximum(h1 + b1_ref[...], 0.0)
    out = jnp.sum(h1 * w2_ref[...], axis=1) + b2_ref[0, 0]
    out_ref[...] = out.reshape(1, 1, _BN)


def _tc_mlp(acc, xs, bm2, ba2, w1m, w1a, w1s, b12, w22, b22):
    grid = (PN // _BN,)
    full = lambda a: pl.BlockSpec(a.shape, lambda i: (0,) * a.ndim)
    return pl.pallas_call(
        _tc_mlp_body,
        grid=grid,
        in_specs=[
            pl.BlockSpec((_BN, 128), lambda i: (i, 0)),
            pl.BlockSpec((_BN, 48), lambda i: (i, 0)),
            full(bm2), full(ba2), full(w1m), full(w1a), full(w1s),
            full(b12), full(w22), full(b22),
        ],
        out_specs=pl.BlockSpec((1, 1, _BN), lambda i: (i, 0, 0)),
        out_shape=jax.ShapeDtypeStruct((PN // _BN, 1, _BN), F32),
    )(acc, xs, bm2, ba2, w1m, w1a, w1s, b12, w22, b22)


def kernel(x, edge_index, edge_attr, W_m, b_m, W_a, b_a, W_s, b_s, W1, b1, W2, b2):
    N = x.shape[0]
    E = edge_attr.shape[0]

    deg0, deg1 = _make_sc_deg(E)(edge_index, edge_attr)

    x_pad = jnp.zeros((PN, x.shape[1]), F32).at[:N].set(x)
    w96 = jnp.concatenate([W_m, W_a, jnp.zeros((32, W_m.shape[1]), F32)], axis=0)
    h96, xs = _tc_h(x_pad, w96, W_s, b_s.reshape(1, 48))
    dinv2 = _tc_dinv(deg0.reshape(PN // _BN, 1, _BN), deg1.reshape(PN // _BN, 1, _BN))
    dinv = dinv2.reshape(PN)

    acc = _make_sc_agg(E)(edge_index, edge_attr, dinv, h96)

    out2 = _tc_mlp(acc, xs,
                   b_m.reshape(1, 48), b_a.reshape(1, 48),
                   W1[:, 0:48].T, W1[:, 48:96].T, W1[:, 96:144].T,
                   b1.reshape(1, 64), W2.reshape(1, 64),
                   b2.reshape(1, 1))
    return out2.reshape(PN)[:N]


# speedup vs baseline: 1.0005x; 1.0005x over previous
import functools

import jax
import jax.numpy as jnp
from jax import lax
from jax.experimental import pallas as pl
from jax.experimental.pallas import tpu as pltpu
from jax.experimental.pallas import tpu_sc as plsc

F32 = jnp.float32
I32 = jnp.int32

NTILES = 32
RPT = 320
PN = NTILES * RPT
CHUNK = 4000
LANES = 16


def _mesh():
    return plsc.VectorSubcoreMesh(core_axis_name="c", subcore_axis_name="s")


def _wid():
    return lax.axis_index("c") * 16 + lax.axis_index("s")


_DSL = 128


def _make_sc_deg(E):
    ept = E // NTILES
    nsl = ept // _DSL
    tail = ept - nsl * _DSL
    nslp = nsl + (1 if tail else 0)

    @functools.partial(
        pl.kernel,
        out_type=(jax.ShapeDtypeStruct((PN,), F32),
                  jax.ShapeDtypeStruct((PN,), F32)),
        mesh=_mesh(),
        compiler_params=pltpu.CompilerParams(needs_layout_passes=False, use_tc_tiling_on_sc=False),
        scratch_types=[
            pltpu.VMEM((nslp, _DSL), I32),
            pltpu.VMEM((nslp, _DSL), F32),
            pltpu.VMEM((1024,), F32),
            pltpu.VMEM_SHARED((PN,), F32),
            pltpu.SemaphoreType.DMA,
            pltpu.SemaphoreType.DMA,
            pltpu.SemaphoreType.DMA,
        ],
    )
    def sc_deg(dst_hbm, ea_hbm, deg0_hbm, deg1_hbm, dstb, eab, zbuf, sdeg,
               seml, semz, sems):
        cid = lax.axis_index("c")
        sid = lax.axis_index("s")
        wid = cid * 16 + sid
        off = wid * ept

        if tail:
            def _zt(i, _):
                dstb[nsl, pl.ds(i * LANES, LANES)] = jnp.zeros((LANES,), I32)
                eab[nsl, pl.ds(i * LANES, LANES)] = jnp.zeros((LANES,), F32)
                return 0
            lax.fori_loop(0, _DSL // LANES, _zt, 0)

        def _ld(j, _):
            pltpu.async_copy(dst_hbm.at[pl.ds(off + j * _DSL, _DSL)], dstb.at[j], seml)
            pltpu.async_copy(ea_hbm.at[pl.ds(off + j * _DSL, _DSL)], eab.at[j], seml)
            return 0
        lax.fori_loop(0, nsl, _ld, 0)
        if tail:
            pltpu.async_copy(dst_hbm.at[pl.ds(off + nsl * _DSL, tail)],
                             dstb.at[nsl, pl.ds(0, tail)], seml)
            pltpu.async_copy(ea_hbm.at[pl.ds(off + nsl * _DSL, tail)],
                             eab.at[nsl, pl.ds(0, tail)], seml)

        def _z(i, _):
            zbuf[pl.ds(i * LANES, LANES)] = jnp.zeros((LANES,), F32)
            return 0
        lax.fori_loop(0, 1024 // LANES, _z, 0)

        @pl.when(sid == 0)
        def _():
            def _zs(k, _):
                pltpu.async_copy(zbuf, sdeg.at[pl.ds(k * 1024, 1024)], semz)
                return 0
            lax.fori_loop(0, PN // 1024, _zs, 0)

            def _zw(k, _):
                pltpu.make_async_copy(zbuf, sdeg.at[pl.ds(k * 1024, 1024)], semz).wait()
                return 0
            lax.fori_loop(0, PN // 1024, _zw, 0)
        plsc.subcore_barrier()

        def _lw(j, _):
            pltpu.make_async_copy(dst_hbm.at[pl.ds(off + j * _DSL, _DSL)], dstb.at[j], seml).wait()
            pltpu.make_async_copy(ea_hbm.at[pl.ds(off + j * _DSL, _DSL)], eab.at[j], seml).wait()
            return 0
        lax.fori_loop(0, nsl, _lw, 0)
        if tail:
            pltpu.make_async_copy(dst_hbm.at[pl.ds(off + nsl * _DSL, tail)],
                                  dstb.at[nsl, pl.ds(0, tail)], seml).wait()
            pltpu.make_async_copy(ea_hbm.at[pl.ds(off + nsl * _DSL, tail)],
                                  eab.at[nsl, pl.ds(0, tail)], seml).wait()

        def _sc(j, _):
            pltpu.async_copy(eab.at[j], sdeg.at[dstb.at[j]], sems, add=True)
            return 0
        lax.fori_loop(0, nslp, _sc, 0)

        def _sw(j, _):
            pltpu.make_async_copy(eab.at[j], sdeg.at[dstb.at[j]], sems).wait()
            return 0
        lax.fori_loop(0, nslp, _sw, 0)

        plsc.subcore_barrier()
        @pl.when(sid == 0)
        def _():
            @pl.when(cid == 0)
            def _():
                pltpu.sync_copy(sdeg, deg0_hbm)

            @pl.when(cid == 1)
            def _():
                pltpu.sync_copy(sdeg, deg1_hbm)

    return sc_deg


_GB = 128


def _make_sc_agg(E):
    nch = E // CHUNK
    nvec = CHUNK // LANES
    ccap = CHUNK + 2 * _GB

    @functools.partial(
        pl.kernel,
        out_type=jax.ShapeDtypeStruct((PN, 128), F32),
        mesh=_mesh(),
        compiler_params=pltpu.CompilerParams(needs_layout_passes=False, use_tc_tiling_on_sc=False),
        scratch_types=[
            pltpu.VMEM((PN + LANES,), F32),
            pltpu.VMEM((2, CHUNK), I32),
            pltpu.VMEM((2, CHUNK), I32),
            pltpu.VMEM((2, CHUNK), F32),
            pltpu.VMEM((ccap,), I32),
            pltpu.VMEM((ccap,), I32),
            pltpu.VMEM((ccap,), F32),
            pltpu.VMEM((LANES, 128), F32),
            pltpu.VMEM((2, _GB, 128), F32),
            pltpu.VMEM((RPT, 128), F32),
            pltpu.SemaphoreType.DMA((2,)),
            pltpu.SemaphoreType.DMA((2,)),
            pltpu.SemaphoreType.DMA,
        ],
    )
    def sc_agg(dst_hbm, src_hbm, ea_hbm, dinv_hbm, h96_hbm, acc_hbm,
               dinv_v, dstb, srcb, eab, src_c, dl_c, nrm_c,
               gbuf, gb2, acc, sems, semg, semt):
        wid = _wid()
        lo = wid * RPT

        def _fire_scan(c):
            p = c % 2
            off = c * CHUNK
            pltpu.async_copy(dst_hbm.at[pl.ds(off, CHUNK)], dstb.at[p], sems.at[p])
            pltpu.async_copy(src_hbm.at[pl.ds(off, CHUNK)], srcb.at[p], sems.at[p])
            pltpu.async_copy(ea_hbm.at[pl.ds(off, CHUNK)], eab.at[p], sems.at[p])

        _fire_scan(0)
        pltpu.sync_copy(dinv_hbm, dinv_v.at[pl.ds(0, PN)])

        def _z(i, _):
            src_c[pl.ds(i * LANES, LANES)] = jnp.zeros((LANES,), I32)
            dl_c[pl.ds(i * LANES, LANES)] = jnp.zeros((LANES,), I32)
            return 0
        lax.fori_loop(0, ccap // LANES, _z, 0)

        for base, blk in ((0, _GB), (_GB, _GB), (2 * _GB, RPT - 2 * _GB)):
            pltpu.sync_copy(h96_hbm.at[pl.ds(lo + base, blk)],
                            gb2.at[0, pl.ds(0, blk)])

            def _init(r, _, base=base):
                dv = dinv_v[pl.ds(lo + base + r, LANES)][0]
                d2 = dv * dv
                for k in range(6):
                    acc[base + r, pl.ds(16 * k, 16)] = \
                        d2 * gb2[0, r, pl.ds(16 * k, 16)]
                return 0
            lax.fori_loop(0, blk, _init, 0)

        def _drain_scan(c):
            p = c % 2
            off = c * CHUNK
            pltpu.make_async_copy(dst_hbm.at[pl.ds(off, CHUNK)], dstb.at[p], sems.at[p]).wait()
            pltpu.make_async_copy(src_hbm.at[pl.ds(off, CHUNK)], srcb.at[p], sems.at[p]).wait()
            pltpu.make_async_copy(ea_hbm.at[pl.ds(off, CHUNK)], eab.at[p], sems.at[p]).wait()

        def _fire_gather(b):
            p = b % 2
            idxs = src_c.at[pl.ds(b * _GB, _GB)]
            pltpu.async_copy(h96_hbm.at[idxs], gb2.at[p], semg.at[p])

        def _drain_gather(b):
            p = b % 2
            pltpu.make_async_copy(h96_hbm.at[src_c.at[pl.ds(0, _GB)]],
                                  gb2.at[p], semg.at[p]).wait()

        def _edge_update(nrm, dl, row_k):
            for k in range(3):
                o = 16 * k
                am = acc[dl, pl.ds(o, 16)]
                acc[dl, pl.ds(o, 16)] = jnp.maximum(am, nrm * row_k(o))
            for k in range(3):
                o = 48 + 16 * k
                aa = acc[dl, pl.ds(o, 16)]
                acc[dl, pl.ds(o, 16)] = aa + nrm * row_k(o)

        def _chunk(c, cnt_in):
            p = c % 2

            @pl.when(c + 1 < nch)
            def _():
                _fire_scan(c + 1)
            _drain_scan(c)

            def _scan(i, carry):
                cnt, fired = carry
                for u in range(2):
                    o = (i * 2 + u) * LANES
                    d = dstb[p, pl.ds(o, LANES)]
                    s = srcb[p, pl.ds(o, LANES)]
                    w = eab[p, pl.ds(o, LANES)]
                    m = (d >= lo) & (d < lo + RPT)
                    plsc.store_compressed(src_c.at[pl.ds(cnt, LANES)], s, mask=m)
                    plsc.store_compressed(dl_c.at[pl.ds(cnt, LANES)], d - lo, mask=m)
                    plsc.store_compressed(nrm_c.at[pl.ds(cnt, LANES)], w, mask=m)
                    cnt = cnt + jnp.sum(m.astype(I32))

                @pl.when((cnt >= _GB) & (fired == 0))
                def _():
                    _fire_gather(0)
                fired = jnp.where(cnt >= _GB, 1, fired)
                return cnt, fired
            cnt, _fired = lax.fori_loop(0, nvec // 2, _scan, (cnt_in, 0))

            full = cnt // _GB

            def _batch(b, _):
                bp = b % 2

                @pl.when(b + 1 < full)
                def _():
                    _fire_gather(b + 1)
                _drain_gather(b)

                def _grp(g, _):
                    e0 = b * _GB + g * LANES
                    sv = src_c[pl.ds(e0, LANES)]
                    dv = dl_c[pl.ds(e0, LANES)]
                    wv = nrm_c[pl.ds(e0, LANES)]
                    nv = (plsc.load_gather(dinv_v, [sv]) * wv
                          * plsc.load_gather(dinv_v, [dv + lo]))
                    for j in range(LANES):
                        _edge_update(nv[j], dv[j],
                                     lambda o, j=j: gb2[bp, g * LANES + j,
                                                        pl.ds(o, 16)])
                    return 0
                lax.fori_loop(0, _GB // LANES, _grp, 0)
                return 0
            lax.fori_loop(0, full, _batch, 0)

            rem = cnt - full * _GB

            @pl.when(full > 0)
            def _():
                done = full * _GB
                for i in range(_GB // LANES):
                    o = i * LANES
                    src_c[pl.ds(o, LANES)] = src_c[pl.ds(done + o, LANES)]
                    dl_c[pl.ds(o, LANES)] = dl_c[pl.ds(done + o, LANES)]
                    nrm_c[pl.ds(o, LANES)] = nrm_c[pl.ds(done + o, LANES)]
            return rem
        rem = lax.fori_loop(0, nch, _chunk, 0)

        def _tailg(g, _):
            idxv = src_c[pl.ds(g * LANES, LANES)]
            pltpu.async_copy(h96_hbm.at[idxv], gbuf, semt).wait()
            ecnt = jnp.minimum(LANES, rem - g * LANES)
            dv = dl_c[pl.ds(g * LANES, LANES)]
            wv = nrm_c[pl.ds(g * LANES, LANES)]
            nv = (plsc.load_gather(dinv_v, [idxv]) * wv
                  * plsc.load_gather(dinv_v, [dv + lo]))
            nrm_c[pl.ds(g * LANES, LANES)] = nv

            def _edge(j, _):
                e = g * LANES + j
                nrm = nrm_c[pl.ds(e, LANES)][0]
                dl = dl_c[pl.ds(e, LANES)][0]
                _edge_update(nrm, dl, lambda o: gbuf[j, pl.ds(o, 16)])
                return 0
            lax.fori_loop(0, ecnt, _edge, 0)
            return 0
        lax.fori_loop(0, (rem + LANES - 1) // LANES, _tailg, 0)

        pltpu.sync_copy(acc, acc_hbm.at[pl.ds(lo, RPT)])

    return sc_agg


_BN = 1024


def _tc_h_body(x_ref, w96_ref, ws_ref, bs_ref, h96_ref, xs_ref):
    xb = x_ref[...]
    h96_ref[...] = lax.dot_general(xb, w96_ref[...], (((1,), (1,)), ((), ())),
                                   preferred_element_type=F32)
    xs = lax.dot_general(xb, ws_ref[...], (((1,), (1,)), ((), ())),
                         preferred_element_type=F32)
    xs_ref[...] = jnp.maximum(xs + bs_ref[...], 0.0)


def _tc_h(x_pad, w96, ws, bs2):
    grid = (PN // _BN,)
    return pl.pallas_call(
        _tc_h_body,
        grid=grid,
        in_specs=[
            pl.BlockSpec((_BN, x_pad.shape[1]), lambda i: (i, 0)),
            pl.BlockSpec(w96.shape, lambda i: (0, 0)),
            pl.BlockSpec(ws.shape, lambda i: (0, 0)),
            pl.BlockSpec(bs2.shape, lambda i: (0, 0)),
        ],
        out_specs=[
            pl.BlockSpec((_BN, 128), lambda i: (i, 0)),
            pl.BlockSpec((_BN, 48), lambda i: (i, 0)),
        ],
        out_shape=[
            jax.ShapeDtypeStruct((PN, 128), F32),
            jax.ShapeDtypeStruct((PN, 48), F32),
        ],
    )(x_pad, w96, ws, bs2)


def _tc_dinv_body(d0_ref, d1_ref, dinv_ref):
    dinv_ref[...] = lax.rsqrt(d0_ref[...] + d1_ref[...] + 1.0)


def _tc_dinv(deg0, deg1):
    grid = (PN // _BN,)
    spec = pl.BlockSpec((1, 1, _BN), lambda i: (i, 0, 0))
    return pl.pallas_call(
        _tc_dinv_body,
        grid=grid,
        in_specs=[spec, spec],
        out_specs=spec,
        out_shape=jax.ShapeDtypeStruct((PN // _BN, 1, _BN), F32),
    )(deg0, deg1)


def _tc_mlp_body(acc_ref, xs_ref, bm_ref, ba_ref,
                 w1m_ref, w1a_ref, w1s_ref, b1_ref, w2_ref, b2_ref, out_ref):
    xm = jnp.maximum(acc_ref[:, 0:48] + bm_ref[...], 0.0)
    xa = jnp.maximum(acc_ref[:, 48:96] + ba_ref[...], 0.0)
    xs = xs_ref[...]
    dn = (((1,), (0,)), ((), ()))
    h1 = lax.dot_general(xm, w1m_ref[...], dn, preferred_element_type=F32)
    h1 = h1 + lax.dot_general(xa, w1a_ref[...], dn, preferred_element_type=F32)
    h1 = h1 + lax.dot_general(xs, w1s_ref[...], dn, preferred_element_type=F32)
    h1 = jnp.maximum(h1 + b1_ref[...], 0.0)
    out = jnp.sum(h1 * w2_ref[...], axis=1) + b2_ref[0, 0]
    out_ref[...] = out.reshape(1, 1, _BN)


def _tc_mlp(acc, xs, bm2, ba2, w1m, w1a, w1s, b12, w22, b22):
    grid = (PN // _BN,)
    full = lambda a: pl.BlockSpec(a.shape, lambda i: (0,) * a.ndim)
    return pl.pallas_call(
        _tc_mlp_body,
        grid=grid,
        in_specs=[
            pl.BlockSpec((_BN, 128), lambda i: (i, 0)),
            pl.BlockSpec((_BN, 48), lambda i: (i, 0)),
            full(bm2), full(ba2), full(w1m), full(w1a), full(w1s),
            full(b12), full(w22), full(b22),
        ],
        out_specs=pl.BlockSpec((1, 1, _BN), lambda i: (i, 0, 0)),
        out_shape=jax.ShapeDtypeStruct((PN // _BN, 1, _BN), F32),
    )(acc, xs, bm2, ba2, w1m, w1a, w1s, b12, w22, b22)


def _tc_split_body(ei_ref, s_ref, d_ref):
    s_ref[...] = ei_ref[0]
    d_ref[...] = ei_ref[1]


def _tc_split(ei):
    E = ei.shape[1]
    return pl.pallas_call(
        _tc_split_body,
        out_shape=[jax.ShapeDtypeStruct((E,), I32),
                   jax.ShapeDtypeStruct((E,), I32)],
    )(ei)


def kernel(x, edge_index, edge_attr, W_m, b_m, W_a, b_a, W_s, b_s, W1, b1, W2, b2):
    N = x.shape[0]
    E = edge_attr.shape[0]

    src, dst = _tc_split(edge_index)
    deg0, deg1 = _make_sc_deg(E)(dst, edge_attr)

    x_pad = jnp.zeros((PN, x.shape[1]), F32).at[:N].set(x)
    w96 = jnp.concatenate([W_m, W_a, jnp.zeros((32, W_m.shape[1]), F32)], axis=0)
    h96, xs = _tc_h(x_pad, w96, W_s, b_s.reshape(1, 48))
    dinv2 = _tc_dinv(deg0.reshape(PN // _BN, 1, _BN), deg1.reshape(PN // _BN, 1, _BN))
    dinv = dinv2.reshape(PN)

    acc = _make_sc_agg(E)(dst, src, edge_attr, dinv, h96)

    out2 = _tc_mlp(acc, xs,
                   b_m.reshape(1, 48), b_a.reshape(1, 48),
                   W1[:, 0:48].T, W1[:, 48:96].T, W1[:, 96:144].T,
                   b1.reshape(1, 64), W2.reshape(1, 64),
                   b2.reshape(1, 1))
    return out2.reshape(PN)[:N]


# speedup vs baseline: 1.0132x; 1.0127x over previous
import functools

import jax
import jax.numpy as jnp
from jax import lax
from jax.experimental import pallas as pl
from jax.experimental.pallas import tpu as pltpu
from jax.experimental.pallas import tpu_sc as plsc

F32 = jnp.float32
I32 = jnp.int32

NTILES = 32
RPT = 320
PN = NTILES * RPT
CHUNK = 4000
LANES = 16


def _mesh():
    return plsc.VectorSubcoreMesh(core_axis_name="c", subcore_axis_name="s")


def _wid():
    return lax.axis_index("c") * 16 + lax.axis_index("s")


_DSL = 128


def _make_sc_deg(E):
    ept = E // NTILES
    nsl = ept // _DSL
    tail = ept - nsl * _DSL
    nslp = nsl + (1 if tail else 0)

    @functools.partial(
        pl.kernel,
        out_type=(jax.ShapeDtypeStruct((PN,), F32),
                  jax.ShapeDtypeStruct((PN,), F32)),
        mesh=_mesh(),
        compiler_params=pltpu.CompilerParams(needs_layout_passes=False, use_tc_tiling_on_sc=False),
        scratch_types=[
            pltpu.VMEM((nslp, _DSL), I32),
            pltpu.VMEM((nslp, _DSL), F32),
            pltpu.VMEM((1024,), F32),
            pltpu.VMEM_SHARED((PN,), F32),
            pltpu.SemaphoreType.DMA,
            pltpu.SemaphoreType.DMA,
            pltpu.SemaphoreType.DMA,
        ],
    )
    def sc_deg(dst_hbm, ea_hbm, deg0_hbm, deg1_hbm, dstb, eab, zbuf, sdeg,
               seml, semz, sems):
        cid = lax.axis_index("c")
        sid = lax.axis_index("s")
        wid = cid * 16 + sid
        off = wid * ept

        if tail:
            def _zt(i, _):
                dstb[nsl, pl.ds(i * LANES, LANES)] = jnp.zeros((LANES,), I32)
                eab[nsl, pl.ds(i * LANES, LANES)] = jnp.zeros((LANES,), F32)
                return 0
            lax.fori_loop(0, _DSL // LANES, _zt, 0)

        def _ld(j, _):
            pltpu.async_copy(dst_hbm.at[pl.ds(off + j * _DSL, _DSL)], dstb.at[j], seml)
            pltpu.async_copy(ea_hbm.at[pl.ds(off + j * _DSL, _DSL)], eab.at[j], seml)
            return 0
        lax.fori_loop(0, nsl, _ld, 0)
        if tail:
            pltpu.async_copy(dst_hbm.at[pl.ds(off + nsl * _DSL, tail)],
                             dstb.at[nsl, pl.ds(0, tail)], seml)
            pltpu.async_copy(ea_hbm.at[pl.ds(off + nsl * _DSL, tail)],
                             eab.at[nsl, pl.ds(0, tail)], seml)

        def _z(i, _):
            zbuf[pl.ds(i * LANES, LANES)] = jnp.zeros((LANES,), F32)
            return 0
        lax.fori_loop(0, 1024 // LANES, _z, 0)

        @pl.when(sid == 0)
        def _():
            def _zs(k, _):
                pltpu.async_copy(zbuf, sdeg.at[pl.ds(k * 1024, 1024)], semz)
                return 0
            lax.fori_loop(0, PN // 1024, _zs, 0)

            def _zw(k, _):
                pltpu.make_async_copy(zbuf, sdeg.at[pl.ds(k * 1024, 1024)], semz).wait()
                return 0
            lax.fori_loop(0, PN // 1024, _zw, 0)
        plsc.subcore_barrier()

        def _lw(j, _):
            pltpu.make_async_copy(dst_hbm.at[pl.ds(off + j * _DSL, _DSL)], dstb.at[j], seml).wait()
            pltpu.make_async_copy(ea_hbm.at[pl.ds(off + j * _DSL, _DSL)], eab.at[j], seml).wait()
            return 0
        lax.fori_loop(0, nsl, _lw, 0)
        if tail:
            pltpu.make_async_copy(dst_hbm.at[pl.ds(off + nsl * _DSL, tail)],
                                  dstb.at[nsl, pl.ds(0, tail)], seml).wait()
            pltpu.make_async_copy(ea_hbm.at[pl.ds(off + nsl * _DSL, tail)],
                                  eab.at[nsl, pl.ds(0, tail)], seml).wait()

        def _sc(j, _):
            pltpu.async_copy(eab.at[j], sdeg.at[dstb.at[j]], sems, add=True)
            return 0
        lax.fori_loop(0, nslp, _sc, 0)

        def _sw(j, _):
            pltpu.make_async_copy(eab.at[j], sdeg.at[dstb.at[j]], sems).wait()
            return 0
        lax.fori_loop(0, nslp, _sw, 0)

        plsc.subcore_barrier()
        @pl.when(sid == 0)
        def _():
            @pl.when(cid == 0)
            def _():
                pltpu.sync_copy(sdeg, deg0_hbm)

            @pl.when(cid == 1)
            def _():
                pltpu.sync_copy(sdeg, deg1_hbm)

    return sc_deg


_GB = 128


def _make_sc_agg(E):
    nch = E // CHUNK
    nvec = CHUNK // LANES
    ccap = CHUNK + 2 * _GB

    @functools.partial(
        pl.kernel,
        out_type=jax.ShapeDtypeStruct((PN, 128), F32),
        mesh=_mesh(),
        compiler_params=pltpu.CompilerParams(needs_layout_passes=False, use_tc_tiling_on_sc=False),
        scratch_types=[
            pltpu.VMEM((PN + LANES,), F32),
            pltpu.VMEM((2, CHUNK), I32),
            pltpu.VMEM((2, CHUNK), I32),
            pltpu.VMEM((2, CHUNK), F32),
            pltpu.VMEM((ccap,), I32),
            pltpu.VMEM((ccap,), I32),
            pltpu.VMEM((ccap,), F32),
            pltpu.VMEM((LANES, 128), F32),
            pltpu.VMEM((2, _GB, 128), F32),
            pltpu.VMEM((RPT, 128), F32),
            pltpu.SemaphoreType.DMA((2,)),
            pltpu.SemaphoreType.DMA((2,)),
            pltpu.SemaphoreType.DMA,
        ],
    )
    def sc_agg(dst_hbm, src_hbm, ea_hbm, dinv_hbm, h96_hbm, acc_hbm,
               dinv_v, dstb, srcb, eab, src_c, dl_c, nrm_c,
               gbuf, gb2, acc, sems, semg, semt):
        wid = _wid()
        lo = wid * RPT

        def _fire_scan(c):
            p = c % 2
            off = c * CHUNK
            pltpu.async_copy(dst_hbm.at[pl.ds(off, CHUNK)], dstb.at[p], sems.at[p])
            pltpu.async_copy(src_hbm.at[pl.ds(off, CHUNK)], srcb.at[p], sems.at[p])
            pltpu.async_copy(ea_hbm.at[pl.ds(off, CHUNK)], eab.at[p], sems.at[p])

        _fire_scan(0)
        pltpu.sync_copy(dinv_hbm, dinv_v.at[pl.ds(0, PN)])

        def _z(i, _):
            src_c[pl.ds(i * LANES, LANES)] = jnp.zeros((LANES,), I32)
            dl_c[pl.ds(i * LANES, LANES)] = jnp.zeros((LANES,), I32)
            return 0
        lax.fori_loop(0, ccap // LANES, _z, 0)

        for base, blk in ((0, _GB), (_GB, _GB), (2 * _GB, RPT - 2 * _GB)):
            pltpu.sync_copy(h96_hbm.at[pl.ds(lo + base, blk)],
                            gb2.at[0, pl.ds(0, blk)])

            def _init(r, _, base=base):
                dv = dinv_v[pl.ds(lo + base + r, LANES)][0]
                d2 = dv * dv
                for k in range(6):
                    acc[base + r, pl.ds(16 * k, 16)] = \
                        d2 * gb2[0, r, pl.ds(16 * k, 16)]
                return 0
            lax.fori_loop(0, blk, _init, 0)

        def _drain_scan(c):
            p = c % 2
            off = c * CHUNK
            pltpu.make_async_copy(dst_hbm.at[pl.ds(off, CHUNK)], dstb.at[p], sems.at[p]).wait()
            pltpu.make_async_copy(src_hbm.at[pl.ds(off, CHUNK)], srcb.at[p], sems.at[p]).wait()
            pltpu.make_async_copy(ea_hbm.at[pl.ds(off, CHUNK)], eab.at[p], sems.at[p]).wait()

        def _fire_gather(b):
            p = b % 2
            idxs = src_c.at[pl.ds(b * _GB, _GB)]
            pltpu.async_copy(h96_hbm.at[idxs], gb2.at[p], semg.at[p])

        def _drain_gather(b):
            p = b % 2
            pltpu.make_async_copy(h96_hbm.at[src_c.at[pl.ds(0, _GB)]],
                                  gb2.at[p], semg.at[p]).wait()

        def _edge_update(nrm, dl, row_k):
            for k in range(3):
                o = 16 * k
                am = acc[dl, pl.ds(o, 16)]
                acc[dl, pl.ds(o, 16)] = jnp.maximum(am, nrm * row_k(o))
            for k in range(3):
                o = 48 + 16 * k
                aa = acc[dl, pl.ds(o, 16)]
                acc[dl, pl.ds(o, 16)] = aa + nrm * row_k(o)

        def _chunk(c, cnt_in):
            p = c % 2

            @pl.when(c + 1 < nch)
            def _():
                _fire_scan(c + 1)
            _drain_scan(c)

            def _scan(i, cnt):
                for u in range(2):
                    o = (i * 2 + u) * LANES
                    d = dstb[p, pl.ds(o, LANES)]
                    s = srcb[p, pl.ds(o, LANES)]
                    w = eab[p, pl.ds(o, LANES)]
                    m = (d >= lo) & (d < lo + RPT)
                    nrm = (plsc.load_gather(dinv_v, [s]) * w
                           * plsc.load_gather(dinv_v, [d]))
                    plsc.store_compressed(src_c.at[pl.ds(cnt, LANES)], s, mask=m)
                    plsc.store_compressed(dl_c.at[pl.ds(cnt, LANES)], d - lo, mask=m)
                    plsc.store_compressed(nrm_c.at[pl.ds(cnt, LANES)], nrm, mask=m)
                    cnt = cnt + jnp.sum(m.astype(I32))
                return cnt
            cnt = lax.fori_loop(0, nvec // 2, _scan, cnt_in)

            full = cnt // _GB

            @pl.when(full > 0)
            def _():
                _fire_gather(0)

            def _batch(b, _):
                bp = b % 2

                @pl.when(b + 1 < full)
                def _():
                    _fire_gather(b + 1)
                _drain_gather(b)

                def _grp(g, _):
                    e0 = b * _GB + g * LANES
                    nv = nrm_c[pl.ds(e0, LANES)]
                    dv = dl_c[pl.ds(e0, LANES)]
                    for j in range(LANES):
                        _edge_update(nv[j], dv[j],
                                     lambda o, j=j: gb2[bp, g * LANES + j,
                                                        pl.ds(o, 16)])
                    return 0
                lax.fori_loop(0, _GB // LANES, _grp, 0)
                return 0
            lax.fori_loop(0, full, _batch, 0)

            rem = cnt - full * _GB

            @pl.when(full > 0)
            def _():
                done = full * _GB
                for i in range(_GB // LANES):
                    o = i * LANES
                    src_c[pl.ds(o, LANES)] = src_c[pl.ds(done + o, LANES)]
                    dl_c[pl.ds(o, LANES)] = dl_c[pl.ds(done + o, LANES)]
                    nrm_c[pl.ds(o, LANES)] = nrm_c[pl.ds(done + o, LANES)]
            return rem
        rem = lax.fori_loop(0, nch, _chunk, 0)

        def _tailg(g, _):
            idxv = src_c[pl.ds(g * LANES, LANES)]
            pltpu.async_copy(h96_hbm.at[idxv], gbuf, semt).wait()
            ecnt = jnp.minimum(LANES, rem - g * LANES)

            def _edge(j, _):
                e = g * LANES + j
                nrm = nrm_c[pl.ds(e, LANES)][0]
                dl = dl_c[pl.ds(e, LANES)][0]
                _edge_update(nrm, dl, lambda o: gbuf[j, pl.ds(o, 16)])
                return 0
            lax.fori_loop(0, ecnt, _edge, 0)
            return 0
        lax.fori_loop(0, (rem + LANES - 1) // LANES, _tailg, 0)

        pltpu.sync_copy(acc, acc_hbm.at[pl.ds(lo, RPT)])

    return sc_agg


_BN = 1024


def _tc_h_body(x_ref, w96_ref, ws_ref, bs_ref, h96_ref, xs_ref):
    xb = x_ref[...]
    h96_ref[...] = lax.dot_general(xb, w96_ref[...], (((1,), (1,)), ((), ())),
                                   preferred_element_type=F32)
    xs = lax.dot_general(xb, ws_ref[...], (((1,), (1,)), ((), ())),
                         preferred_element_type=F32)
    xs_ref[...] = jnp.maximum(xs + bs_ref[...], 0.0)


def _tc_h(x_pad, w96, ws, bs2):
    grid = (PN // _BN,)
    return pl.pallas_call(
        _tc_h_body,
        grid=grid,
        in_specs=[
            pl.BlockSpec((_BN, x_pad.shape[1]), lambda i: (i, 0)),
            pl.BlockSpec(w96.shape, lambda i: (0, 0)),
            pl.BlockSpec(ws.shape, lambda i: (0, 0)),
            pl.BlockSpec(bs2.shape, lambda i: (0, 0)),
        ],
        out_specs=[
            pl.BlockSpec((_BN, 128), lambda i: (i, 0)),
            pl.BlockSpec((_BN, 48), lambda i: (i, 0)),
        ],
        out_shape=[
            jax.ShapeDtypeStruct((PN, 128), F32),
            jax.ShapeDtypeStruct((PN, 48), F32),
        ],
    )(x_pad, w96, ws, bs2)


def _tc_dinv_body(d0_ref, d1_ref, dinv_ref):
    dinv_ref[...] = lax.rsqrt(d0_ref[...] + d1_ref[...] + 1.0)


def _tc_dinv(deg0, deg1):
    grid = (PN // _BN,)
    spec = pl.BlockSpec((1, 1, _BN), lambda i: (i, 0, 0))
    return pl.pallas_call(
        _tc_dinv_body,
        grid=grid,
        in_specs=[spec, spec],
        out_specs=spec,
        out_shape=jax.ShapeDtypeStruct((PN // _BN, 1, _BN), F32),
    )(deg0, deg1)


def _tc_mlp_body(acc_ref, xs_ref, bm_ref, ba_ref,
                 w1m_ref, w1a_ref, w1s_ref, b1_ref, w2_ref, b2_ref, out_ref):
    xm = jnp.maximum(acc_ref[:, 0:48] + bm_ref[...], 0.0)
    xa = jnp.maximum(acc_ref[:, 48:96] + ba_ref[...], 0.0)
    xs = xs_ref[...]
    dn = (((1,), (0,)), ((), ()))
    h1 = lax.dot_general(xm, w1m_ref[...], dn, preferred_element_type=F32)
    h1 = h1 + lax.dot_general(xa, w1a_ref[...], dn, preferred_element_type=F32)
    h1 = h1 + lax.dot_general(xs, w1s_ref[...], dn, preferred_element_type=F32)
    h1 = jnp.maximum(h1 + b1_ref[...], 0.0)
    out = jnp.sum(h1 * w2_ref[...], axis=1) + b2_ref[0, 0]
    out_ref[...] = out.reshape(1, 1, _BN)


def _tc_mlp(acc, xs, bm2, ba2, w1m, w1a, w1s, b12, w22, b22):
    grid = (PN // _BN,)
    full = lambda a: pl.BlockSpec(a.shape, lambda i: (0,) * a.ndim)
    return pl.pallas_call(
        _tc_mlp_body,
        grid=grid,
        in_specs=[
            pl.BlockSpec((_BN, 128), lambda i: (i, 0)),
            pl.BlockSpec((_BN, 48), lambda i: (i, 0)),
            full(bm2), full(ba2), full(w1m), full(w1a), full(w1s),
            full(b12), full(w22), full(b22),
        ],
        out_specs=pl.BlockSpec((1, 1, _BN), lambda i: (i, 0, 0)),
        out_shape=jax.ShapeDtypeStruct((PN // _BN, 1, _BN), F32),
    )(acc, xs, bm2, ba2, w1m, w1a, w1s, b12, w22, b22)


def _tc_split_body(ei_ref, s_ref, d_ref):
    s_ref[...] = ei_ref[0]
    d_ref[...] = ei_ref[1]


def _tc_split(ei):
    E = ei.shape[1]
    return pl.pallas_call(
        _tc_split_body,
        out_shape=[jax.ShapeDtypeStruct((E,), I32),
                   jax.ShapeDtypeStruct((E,), I32)],
    )(ei)


def kernel(x, edge_index, edge_attr, W_m, b_m, W_a, b_a, W_s, b_s, W1, b1, W2, b2):
    N = x.shape[0]
    E = edge_attr.shape[0]

    src, dst = _tc_split(edge_index)
    deg0, deg1 = _make_sc_deg(E)(dst, edge_attr)

    x_pad = jnp.zeros((PN, x.shape[1]), F32).at[:N].set(x)
    w96 = jnp.concatenate([W_m, W_a, jnp.zeros((32, W_m.shape[1]), F32)], axis=0)
    h96, xs = _tc_h(x_pad, w96, W_s, b_s.reshape(1, 48))
    dinv2 = _tc_dinv(deg0.reshape(PN // _BN, 1, _BN), deg1.reshape(PN // _BN, 1, _BN))
    dinv = dinv2.reshape(PN)

    acc = _make_sc_agg(E)(dst, src, edge_attr, dinv, h96)

    out2 = _tc_mlp(acc, xs,
                   b_m.reshape(1, 48), b_a.reshape(1, 48),
                   W1[:, 0:48].T, W1[:, 48:96].T, W1[:, 96:144].T,
                   b1.reshape(1, 64), W2.reshape(1, 64),
                   b2.reshape(1, 1))
    return out2.reshape(PN)[:N]
